# baseline (speedup 1.0000x reference)
import functools

import jax
import jax.numpy as jnp
from jax import lax
from jax.experimental import pallas as pl
from jax.experimental.pallas import tpu as pltpu
from jax.experimental.pallas import tpu_sc as plsc

N = 10000
D = 128
E = 320000
NPAD = 10240
HR = NPAD // 128
NS = 16
EPT = E // NS
CH = 128
NCH = EPT // CH
REM = EPT - NCH * CH
CPT = 160
SPT = CPT * CH
PADE = NS * SPT - E
TRASH = N

_mesh = plsc.VectorSubcoreMesh(core_axis_name="c", subcore_axis_name="s")
_sc_params = pltpu.CompilerParams(needs_layout_passes=False)


@functools.partial(
    pl.kernel,
    out_type=jax.ShapeDtypeStruct((2 * 2 * HR, 128), jnp.float32),
    mesh=_mesh,
    scratch_types=[
        pltpu.VMEM((EPT,), jnp.int32),
        pltpu.VMEM((EPT,), jnp.int32),
        pltpu.VMEM((2 * HR, 128), jnp.float32),
        pltpu.VMEM((HR,), jnp.int32),
        pltpu.VMEM((HR,), jnp.int32),
        pltpu.VMEM_SHARED((2 * HR, 128), jnp.float32),
    ],
    compiler_params=_sc_params,
)
def _deg_kernel(edges_hbm, out_hbm, sidx, didx, hist, rs, rd, shist):
    c = lax.axis_index("c")
    s = lax.axis_index("s")
    zero16 = jnp.zeros((16,), jnp.float32)

    def zbody(i, _):
        for k in range(8):
            hist[i, pl.ds(k * 16, 16)] = zero16
        return 0
    lax.fori_loop(0, 2 * HR, zbody, 0)

    iota = lax.iota(jnp.int32, 16)
    for j in range(HR // 16):
        rs[pl.ds(j * 16, 16)] = iota + (j * 16)
        rd[pl.ds(j * 16, 16)] = iota + (HR + j * 16)

    @pl.when(s == 0)
    def _():
        pltpu.sync_copy(hist, shist)

    sbase = (2 * c) * E + s * EPT
    dbase = (2 * c + 1) * E + s * EPT
    pltpu.sync_copy(edges_hbm.at[pl.ds(sbase, EPT)], sidx)
    pltpu.sync_copy(edges_hbm.at[pl.ds(dbase, EPT)], didx)

    ones = jnp.ones((16,), jnp.float32)

    def count(idx_ref, row_off):
        def body(i, _):
            for k in range(4):
                v = idx_ref[pl.ds(i * 64 + k * 16, 16)]
                r = lax.shift_right_logical(v, 7) + row_off
                col = jnp.bitwise_and(v, 127)
                plsc.addupdate_scatter(hist, [r, col], ones)
            return 0
        lax.fori_loop(0, EPT // 64, body, 0)

    count(sidx, 0)
    count(didx, HR)
    plsc.subcore_barrier()
    pltpu.sync_copy(hist.at[pl.ds(0, HR)], shist.at[rs], add=True)
    pltpu.sync_copy(hist.at[pl.ds(HR, HR)], shist.at[rd], add=True)
    plsc.subcore_barrier()

    @pl.when(s < 10)
    def _():
        pltpu.sync_copy(shist.at[pl.ds(s * 16, 16)],
                        out_hbm.at[pl.ds(c * (2 * HR) + s * 16, 16)])


@functools.partial(
    pl.kernel,
    out_type=jax.ShapeDtypeStruct((2 * N, 128), jnp.float32),
    mesh=_mesh,
    scratch_types=[
        pltpu.VMEM((CH,), jnp.int32),
        pltpu.VMEM((CH,), jnp.int32),
        pltpu.VMEM((CH,), jnp.int32),
        pltpu.VMEM((CH,), jnp.int32),
        pltpu.VMEM((CH, 128), jnp.float32),
        pltpu.VMEM((CH, 128), jnp.float32),
        pltpu.VMEM((16, 128), jnp.float32),
        pltpu.VMEM_SHARED((N + 8, 128), jnp.float32),
        pltpu.SemaphoreType.DMA,
        pltpu.SemaphoreType.DMA,
        pltpu.SemaphoreType.DMA,
        pltpu.SemaphoreType.DMA,
    ],
    compiler_params=_sc_params,
)
def _agg_kernel(src_hbm, dst_hbm, msg_hbm, out_hbm,
                sidx0, didx0, sidx1, didx1, r0, r1, zbuf, agg,
                si0, si1, sg0, sg1):
    c = lax.axis_index("c")
    s = lax.axis_index("s")

    ebase = (c * NS + s) * SPT
    sidx = (sidx0, sidx1)
    didx = (didx0, didx1)
    sem_i = (si0, si1)
    rows = (r0, r1)
    sem_g = (sg0, sg1)

    def start_idx(k, b):
        off = ebase + k * CH
        pltpu.async_copy(src_hbm.at[pl.ds(off, CH)], sidx[b], sem_i[b])
        pltpu.async_copy(dst_hbm.at[pl.ds(off, CH)], didx[b], sem_i[b])

    def wait_idx(b):
        pltpu.make_async_copy(src_hbm.at[pl.ds(0, CH)], sidx[b], sem_i[b]).wait()
        pltpu.make_async_copy(dst_hbm.at[pl.ds(0, CH)], didx[b], sem_i[b]).wait()

    def sg(b):
        pltpu.async_copy(msg_hbm.at[sidx[b]], rows[b], sem_g[b])

    def wg(b):
        pltpu.make_async_copy(msg_hbm.at[sidx[b]], rows[b], sem_g[b]).wait()

    def ssync(b):
        pltpu.sync_copy(rows[b], agg.at[didx[b]], add=True)

    start_idx(0, 0)
    start_idx(1, 1)

    zero16 = jnp.zeros((16,), jnp.float32)
    for r in range(16):
        for k in range(8):
            zbuf[r, pl.ds(k * 16, 16)] = zero16

    rpt = 624
    zb_base = s * rpt

    def zb(i, _):
        pltpu.sync_copy(zbuf, agg.at[pl.ds(zb_base + i * 16, 16)])
        return 0
    lax.fori_loop(0, rpt // 16, zb, 0)

    @pl.when(s == 0)
    def _():
        pltpu.sync_copy(zbuf, agg.at[pl.ds(NS * rpt, 16)])
    plsc.subcore_barrier()

    def ssync(b):
        pltpu.sync_copy(rows[b], agg.at[didx[b]], add=True)

    wait_idx(0)
    sg(0)

    def pipe(j, _):
        i = 2 * j
        wg(0)
        wait_idx(1)
        sg(1)
        ssync(0)
        start_idx(i + 2, 0)
        wg(1)
        wait_idx(0)
        sg(0)
        ssync(1)
        start_idx(i + 3, 1)
        return 0
    lax.fori_loop(0, CPT // 2 - 1, pipe, 0)

    wg(0)
    wait_idx(1)
    sg(1)
    ssync(0)
    wg(1)
    ssync(1)

    plsc.subcore_barrier()
    pltpu.sync_copy(agg.at[pl.ds(s * rpt, rpt)],
                    out_hbm.at[pl.ds(c * N + s * rpt, rpt)])

    @pl.when(s == 0)
    def _():
        pltpu.sync_copy(agg.at[pl.ds(NS * rpt, 16)],
                        out_hbm.at[pl.ds(c * N + NS * rpt, 16)])


def _norm(deg):
    return jnp.where(deg > 0, lax.rsqrt(jnp.maximum(deg, 1e-12)), 0.0)


def _msg_body(x_ref, degs_ref, out_ref):
    x = x_ref[...]
    for m in range(2):
        norm = _norm(degs_ref[m, 0, :])
        out_ref[m] = x * norm[:N, None]


def _stats_body(agg_ref, degs_ref, w1_ref, b1_ref, w2t_ref, beta_ref):
    acc = []
    for m in range(2):
        norm = _norm(degs_ref[m, 1, :])
        h = agg_ref[m] * norm[:N, None]
        t = jnp.tanh(
            jnp.dot(h, w1_ref[...], preferred_element_type=jnp.float32)
            + b1_ref[...][None, :])
        acc.append(jnp.sum(t * w2t_ref[...]) / N)
    w0, w1 = acc
    mx = jnp.maximum(w0, w1)
    e0 = jnp.exp(w0 - mx)
    e1 = jnp.exp(w1 - mx)
    beta_ref[0] = e0 / (e0 + e1)
    beta_ref[1] = e1 / (e0 + e1)


def _comb_body(agg_ref, degs_ref, beta_ref, out_ref):
    acc = None
    for m in range(2):
        norm = _norm(degs_ref[m, 1, :])
        term = (agg_ref[m] * norm[:N, None]) * beta_ref[m]
        acc = term if acc is None else acc + term
    out_ref[...] = acc


def kernel(x, edge_index_0, edge_index_1, W1, b1, W2):
    edges = jnp.concatenate(
        [edge_index_0[0], edge_index_0[1], edge_index_1[0], edge_index_1[1]])
    pad_src = jnp.zeros((PADE,), jnp.int32)
    pad_dst = jnp.full((PADE,), TRASH, jnp.int32)
    src_agg = jnp.concatenate(
        [edge_index_0[0], pad_src, edge_index_1[0] + N, pad_src + N])
    dst_agg = jnp.concatenate(
        [edge_index_0[1], pad_dst, edge_index_1[1], pad_dst])
    degs = _deg_kernel(edges).reshape(2, 2, NPAD)
    msg = pl.pallas_call(
        _msg_body,
        out_shape=jax.ShapeDtypeStruct((2, N, D), jnp.float32),
    )(x, degs)
    agg = _agg_kernel(src_agg, dst_agg, msg.reshape(2 * N, D)).reshape(2, N, D)
    beta = pl.pallas_call(
        _stats_body,
        out_shape=jax.ShapeDtypeStruct((2,), jnp.float32),
        out_specs=pl.BlockSpec(memory_space=pltpu.SMEM),
    )(agg, degs, W1, b1, W2.T)
    out = pl.pallas_call(
        _comb_body,
        in_specs=[
            pl.BlockSpec(memory_space=pltpu.VMEM),
            pl.BlockSpec(memory_space=pltpu.VMEM),
            pl.BlockSpec(memory_space=pltpu.SMEM),
        ],
        out_shape=jax.ShapeDtypeStruct((N, D), jnp.float32),
    )(agg, degs, beta)
    return out


# speedup vs baseline: 2.1049x; 2.1049x over previous
import functools

import jax
import jax.numpy as jnp
from jax import lax
from jax.experimental import pallas as pl
from jax.experimental.pallas import tpu as pltpu
from jax.experimental.pallas import tpu_sc as plsc

N = 10000
D = 128
E = 320000
NPAD = 10240
HR = NPAD // 128
NS = 16
EPT = E // NS
CH = 128
NCH = EPT // CH
REM = EPT - NCH * CH
CPT = 160
SPT = CPT * CH
PADE = NS * SPT - E
TRASH = N

_mesh = plsc.VectorSubcoreMesh(core_axis_name="c", subcore_axis_name="s")
_sc_params = pltpu.CompilerParams(needs_layout_passes=False)


@functools.partial(
    pl.kernel,
    out_type=jax.ShapeDtypeStruct((2 * 2 * HR, 128), jnp.float32),
    mesh=_mesh,
    scratch_types=[
        pltpu.VMEM((EPT,), jnp.int32),
        pltpu.VMEM((EPT,), jnp.int32),
        pltpu.VMEM((2 * HR, 128), jnp.float32),
        pltpu.VMEM((HR,), jnp.int32),
        pltpu.VMEM((HR,), jnp.int32),
        pltpu.VMEM_SHARED((2 * HR, 128), jnp.float32),
    ],
    compiler_params=_sc_params,
)
def _deg_kernel(edges_hbm, out_hbm, sidx, didx, hist, rs, rd, shist):
    c = lax.axis_index("c")
    s = lax.axis_index("s")
    zero16 = jnp.zeros((16,), jnp.float32)

    def zbody(i, _):
        for k in range(8):
            hist[i, pl.ds(k * 16, 16)] = zero16
        return 0
    lax.fori_loop(0, 2 * HR, zbody, 0)

    iota = lax.iota(jnp.int32, 16)
    for j in range(HR // 16):
        rs[pl.ds(j * 16, 16)] = iota + (j * 16)
        rd[pl.ds(j * 16, 16)] = iota + (HR + j * 16)

    @pl.when(s == 0)
    def _():
        pltpu.sync_copy(hist, shist)

    sbase = (2 * c) * E + s * EPT
    dbase = (2 * c + 1) * E + s * EPT
    pltpu.sync_copy(edges_hbm.at[pl.ds(sbase, EPT)], sidx)
    pltpu.sync_copy(edges_hbm.at[pl.ds(dbase, EPT)], didx)

    ones = jnp.ones((16,), jnp.float32)

    def count(idx_ref, row_off):
        def body(i, _):
            for k in range(4):
                v = idx_ref[pl.ds(i * 64 + k * 16, 16)]
                r = lax.shift_right_logical(v, 7) + row_off
                col = jnp.bitwise_and(v, 127)
                plsc.addupdate_scatter(hist, [r, col], ones)
            return 0
        lax.fori_loop(0, EPT // 64, body, 0)

    count(sidx, 0)
    count(didx, HR)
    plsc.subcore_barrier()
    pltpu.sync_copy(hist.at[pl.ds(0, HR)], shist.at[rs], add=True)
    pltpu.sync_copy(hist.at[pl.ds(HR, HR)], shist.at[rd], add=True)
    plsc.subcore_barrier()

    @pl.when(s < 10)
    def _():
        pltpu.sync_copy(shist.at[pl.ds(s * 16, 16)],
                        out_hbm.at[pl.ds(c * (2 * HR) + s * 16, 16)])


@functools.partial(
    pl.kernel,
    out_type=jax.ShapeDtypeStruct((2 * N, 128), jnp.float32),
    mesh=_mesh,
    scratch_types=[
        pltpu.VMEM((CH,), jnp.int32),
        pltpu.VMEM((CH,), jnp.int32),
        pltpu.VMEM((CH,), jnp.int32),
        pltpu.VMEM((CH,), jnp.int32),
        pltpu.VMEM((CH, 128), jnp.float32),
        pltpu.VMEM((CH, 128), jnp.float32),
        pltpu.VMEM((16, 128), jnp.float32),
        pltpu.VMEM_SHARED((N + 128, 128), jnp.float32),
        pltpu.SemaphoreType.DMA,
        pltpu.SemaphoreType.DMA,
        pltpu.SemaphoreType.DMA,
        pltpu.SemaphoreType.DMA,
    ],
    compiler_params=_sc_params,
)
def _agg_kernel(src_hbm, dst_hbm, msg_hbm, out_hbm,
                sidx0, didx0, sidx1, didx1, r0, r1, zbuf, agg,
                si0, si1, sg0, sg1):
    c = lax.axis_index("c")
    s = lax.axis_index("s")

    ebase = (c * NS + s) * SPT
    sidx = (sidx0, sidx1)
    didx = (didx0, didx1)
    sem_i = (si0, si1)
    rows = (r0, r1)
    sem_g = (sg0, sg1)

    def start_idx(k, b):
        off = ebase + k * CH
        pltpu.async_copy(src_hbm.at[pl.ds(off, CH)], sidx[b], sem_i[b])
        pltpu.async_copy(dst_hbm.at[pl.ds(off, CH)], didx[b], sem_i[b])

    def wait_idx(b):
        pltpu.make_async_copy(src_hbm.at[pl.ds(0, CH)], sidx[b], sem_i[b]).wait()
        pltpu.make_async_copy(dst_hbm.at[pl.ds(0, CH)], didx[b], sem_i[b]).wait()

    def sg(b):
        pltpu.async_copy(msg_hbm.at[sidx[b]], rows[b], sem_g[b])

    def wg(b):
        pltpu.make_async_copy(msg_hbm.at[sidx[b]], rows[b], sem_g[b]).wait()

    def ssync(b):
        pltpu.sync_copy(rows[b], agg.at[didx[b]], add=True)

    start_idx(0, 0)
    start_idx(1, 1)

    zero16 = jnp.zeros((16,), jnp.float32)
    for r in range(16):
        for k in range(8):
            zbuf[r, pl.ds(k * 16, 16)] = zero16

    rpt = 624
    zb_base = s * rpt

    def zb(i, _):
        pltpu.sync_copy(zbuf, agg.at[pl.ds(zb_base + i * 16, 16)])
        return 0
    lax.fori_loop(0, rpt // 16, zb, 0)

    @pl.when(s == 0)
    def _():
        pltpu.sync_copy(zbuf, agg.at[pl.ds(NS * rpt, 16)])
    plsc.subcore_barrier()

    def ssync(b):
        pltpu.sync_copy(rows[b], agg.at[didx[b]], add=True)

    wait_idx(0)
    sg(0)

    def pipe(j, _):
        i = 2 * j
        wg(0)
        wait_idx(1)
        sg(1)
        ssync(0)
        start_idx(i + 2, 0)
        wg(1)
        wait_idx(0)
        sg(0)
        ssync(1)
        start_idx(i + 3, 1)
        return 0
    lax.fori_loop(0, CPT // 2 - 1, pipe, 0)

    wg(0)
    wait_idx(1)
    sg(1)
    ssync(0)
    wg(1)
    ssync(1)

    plsc.subcore_barrier()
    pltpu.sync_copy(agg.at[pl.ds(s * rpt, rpt)],
                    out_hbm.at[pl.ds(c * N + s * rpt, rpt)])

    @pl.when(s == 0)
    def _():
        pltpu.sync_copy(agg.at[pl.ds(NS * rpt, 16)],
                        out_hbm.at[pl.ds(c * N + NS * rpt, 16)])


def _norm(deg):
    return jnp.where(deg > 0, lax.rsqrt(jnp.maximum(deg, 1e-12)), 0.0)


def _msg_body(x_ref, degs_ref, out_ref):
    x = x_ref[...]
    for m in range(2):
        norm = _norm(degs_ref[m, 0, :])
        out_ref[m] = x * norm[:N, None]


def _stats_body(agg_ref, degs_ref, w1_ref, b1_ref, w2t_ref, beta_ref):
    acc = []
    for m in range(2):
        norm = _norm(degs_ref[m, 1, :])
        h = agg_ref[m] * norm[:N, None]
        t = jnp.tanh(
            jnp.dot(h, w1_ref[...], preferred_element_type=jnp.float32)
            + b1_ref[...][None, :])
        acc.append(jnp.sum(t * w2t_ref[...]) / N)
    w0, w1 = acc
    mx = jnp.maximum(w0, w1)
    e0 = jnp.exp(w0 - mx)
    e1 = jnp.exp(w1 - mx)
    beta_ref[0] = e0 / (e0 + e1)
    beta_ref[1] = e1 / (e0 + e1)


def _comb_body(agg_ref, degs_ref, beta_ref, out_ref):
    acc = None
    for m in range(2):
        norm = _norm(degs_ref[m, 1, :])
        term = (agg_ref[m] * norm[:N, None]) * beta_ref[m]
        acc = term if acc is None else acc + term
    out_ref[...] = acc


def kernel(x, edge_index_0, edge_index_1, W1, b1, W2):
    edges = jnp.concatenate(
        [edge_index_0[0], edge_index_0[1], edge_index_1[0], edge_index_1[1]])
    ar = jnp.arange(PADE, dtype=jnp.int32)
    pad_src = ar % N
    pad_dst = TRASH + (ar % 128)
    src_agg = jnp.concatenate(
        [edge_index_0[0], pad_src, edge_index_1[0] + N, pad_src + N])
    dst_agg = jnp.concatenate(
        [edge_index_0[1], pad_dst, edge_index_1[1], pad_dst])
    degs = _deg_kernel(edges).reshape(2, 2, NPAD)
    msg = pl.pallas_call(
        _msg_body,
        out_shape=jax.ShapeDtypeStruct((2, N, D), jnp.float32),
    )(x, degs)
    agg = _agg_kernel(src_agg, dst_agg, msg.reshape(2 * N, D)).reshape(2, N, D)
    beta = pl.pallas_call(
        _stats_body,
        out_shape=jax.ShapeDtypeStruct((2,), jnp.float32),
        out_specs=pl.BlockSpec(memory_space=pltpu.SMEM),
    )(agg, degs, W1, b1, W2.T)
    out = pl.pallas_call(
        _comb_body,
        in_specs=[
            pl.BlockSpec(memory_space=pltpu.VMEM),
            pl.BlockSpec(memory_space=pltpu.VMEM),
            pl.BlockSpec(memory_space=pltpu.SMEM),
        ],
        out_shape=jax.ShapeDtypeStruct((N, D), jnp.float32),
    )(agg, degs, beta)
    return out


# speedup vs baseline: 2.6756x; 1.2711x over previous
import functools

import jax
import jax.numpy as jnp
from jax import lax
from jax.experimental import pallas as pl
from jax.experimental.pallas import tpu as pltpu
from jax.experimental.pallas import tpu_sc as plsc

N = 10000
D = 128
E = 320000
NPAD = 10240
HR = NPAD // 128
NS = 16
EPT = E // NS
CH = 96
CPT = 210
SPT = CPT * CH
PADE = NS * SPT - E
TRASH = N

_mesh = plsc.VectorSubcoreMesh(core_axis_name="c", subcore_axis_name="s")
_sc_params = pltpu.CompilerParams(needs_layout_passes=False)


@functools.partial(
    pl.kernel,
    out_type=jax.ShapeDtypeStruct((2 * 2 * HR, 128), jnp.float32),
    mesh=_mesh,
    scratch_types=[
        pltpu.VMEM((EPT,), jnp.int32),
        pltpu.VMEM((EPT,), jnp.int32),
        pltpu.VMEM((2 * HR, 128), jnp.float32),
        pltpu.VMEM((HR,), jnp.int32),
        pltpu.VMEM((HR,), jnp.int32),
        pltpu.VMEM_SHARED((2 * HR, 128), jnp.float32),
    ],
    compiler_params=_sc_params,
)
def _deg_kernel(edges_hbm, out_hbm, sidx, didx, hist, rs, rd, shist):
    c = lax.axis_index("c")
    s = lax.axis_index("s")
    zero16 = jnp.zeros((16,), jnp.float32)

    def zbody(i, _):
        for k in range(8):
            hist[i, pl.ds(k * 16, 16)] = zero16
        return 0
    lax.fori_loop(0, 2 * HR, zbody, 0)

    iota = lax.iota(jnp.int32, 16)
    for j in range(HR // 16):
        rs[pl.ds(j * 16, 16)] = iota + (j * 16)
        rd[pl.ds(j * 16, 16)] = iota + (HR + j * 16)

    @pl.when(s == 0)
    def _():
        pltpu.sync_copy(hist, shist)

    sbase = (2 * c) * E + s * EPT
    dbase = (2 * c + 1) * E + s * EPT
    pltpu.sync_copy(edges_hbm.at[pl.ds(sbase, EPT)], sidx)
    pltpu.sync_copy(edges_hbm.at[pl.ds(dbase, EPT)], didx)

    ones = jnp.ones((16,), jnp.float32)

    def count(idx_ref, row_off):
        def body(i, _):
            for k in range(4):
                v = idx_ref[pl.ds(i * 64 + k * 16, 16)]
                r = lax.shift_right_logical(v, 7) + row_off
                col = jnp.bitwise_and(v, 127)
                plsc.addupdate_scatter(hist, [r, col], ones)
            return 0
        lax.fori_loop(0, EPT // 64, body, 0)

    count(sidx, 0)
    count(didx, HR)
    plsc.subcore_barrier()
    pltpu.sync_copy(hist.at[pl.ds(0, HR)], shist.at[rs], add=True)
    pltpu.sync_copy(hist.at[pl.ds(HR, HR)], shist.at[rd], add=True)
    plsc.subcore_barrier()

    @pl.when(s < 10)
    def _():
        pltpu.sync_copy(shist.at[pl.ds(s * 16, 16)],
                        out_hbm.at[pl.ds(c * (2 * HR) + s * 16, 16)])


@functools.partial(
    pl.kernel,
    out_type=jax.ShapeDtypeStruct((2 * N, 128), jnp.float32),
    mesh=_mesh,
    scratch_types=[
        [pltpu.VMEM((CH,), jnp.int32)] * 6,
        [pltpu.VMEM((CH,), jnp.int32)] * 6,
        [pltpu.VMEM((CH, 128), jnp.float32)] * 3,
        pltpu.VMEM((16, 128), jnp.float32),
        pltpu.VMEM_SHARED((N + 32, 128), jnp.float32),
        [pltpu.SemaphoreType.DMA] * 6,
        [pltpu.SemaphoreType.DMA] * 3,
    ],
    compiler_params=_sc_params,
)
def _agg_kernel(src_hbm, dst_hbm, msg_hbm, out_hbm,
                sidx, didx, rows, zbuf, agg, sem_i, sem_g):
    c = lax.axis_index("c")
    s = lax.axis_index("s")

    ebase = (c * NS + s) * SPT

    def start_idx(k, p):
        off = ebase + k * CH
        pltpu.async_copy(src_hbm.at[pl.ds(off, CH)], sidx[p], sem_i[p])
        pltpu.async_copy(dst_hbm.at[pl.ds(off, CH)], didx[p], sem_i[p])

    def wait_idx(p):
        pltpu.make_async_copy(src_hbm.at[pl.ds(0, CH)], sidx[p], sem_i[p]).wait()
        pltpu.make_async_copy(dst_hbm.at[pl.ds(0, CH)], didx[p], sem_i[p]).wait()

    def sg(p, b):
        pltpu.async_copy(msg_hbm.at[sidx[p]], rows[b], sem_g[b])

    def wg(b):
        pltpu.make_async_copy(msg_hbm.at[sidx[0]], rows[b], sem_g[b]).wait()

    def ssync(p, b):
        pltpu.sync_copy(rows[b], agg.at[didx[p]], add=True)

    for p in range(6):
        start_idx(p, p)

    zero16 = jnp.zeros((16,), jnp.float32)
    for r in range(16):
        for k in range(8):
            zbuf[r, pl.ds(k * 16, 16)] = zero16

    rpt = 624
    zb_base = s * rpt

    def zb(i, _):
        pltpu.sync_copy(zbuf, agg.at[pl.ds(zb_base + i * 16, 16)])
        return 0
    lax.fori_loop(0, rpt // 16, zb, 0)

    @pl.when(s == 0)
    def _():
        pltpu.sync_copy(zbuf, agg.at[pl.ds(NS * rpt, 16)])
    plsc.subcore_barrier()

    wait_idx(0)
    sg(0, 0)
    wait_idx(1)
    sg(1, 1)

    NJ = CPT // 6
    def pipe(j, _):
        k = 6 * j
        for m in range(6):
            pg = (m + 2) % 6
            if m < 4:
                wait_idx(pg)
                sg(pg, (m + 2) % 3)
            else:
                @pl.when(j < NJ - 1)
                def _(pg=pg, m=m):
                    wait_idx(pg)
                    sg(pg, (m + 2) % 3)
            wg(m % 3)
            ssync(m, m % 3)

            @pl.when(j < NJ - 1)
            def _(m=m):
                start_idx(k + m + 6, m)
        return 0
    lax.fori_loop(0, NJ, pipe, 0)

    plsc.subcore_barrier()
    pltpu.sync_copy(agg.at[pl.ds(s * rpt, rpt)],
                    out_hbm.at[pl.ds(c * N + s * rpt, rpt)])

    @pl.when(s == 0)
    def _():
        pltpu.sync_copy(agg.at[pl.ds(NS * rpt, 16)],
                        out_hbm.at[pl.ds(c * N + NS * rpt, 16)])


def _norm(deg):
    return jnp.where(deg > 0, lax.rsqrt(jnp.maximum(deg, 1e-12)), 0.0)


def _msg_body(x_ref, degs_ref, out_ref):
    x = x_ref[...]
    for m in range(2):
        norm = _norm(degs_ref[m, 0, :])
        out_ref[m] = x * norm[:N, None]


def _stats_body(agg_ref, degs_ref, w1_ref, b1_ref, w2t_ref, beta_ref):
    acc = []
    for m in range(2):
        norm = _norm(degs_ref[m, 1, :])
        h = agg_ref[m] * norm[:N, None]
        t = jnp.tanh(
            jnp.dot(h, w1_ref[...], preferred_element_type=jnp.float32)
            + b1_ref[...][None, :])
        acc.append(jnp.sum(t * w2t_ref[...]) / N)
    w0, w1 = acc
    mx = jnp.maximum(w0, w1)
    e0 = jnp.exp(w0 - mx)
    e1 = jnp.exp(w1 - mx)
    beta_ref[0] = e0 / (e0 + e1)
    beta_ref[1] = e1 / (e0 + e1)


def _comb_body(agg_ref, degs_ref, beta_ref, out_ref):
    acc = None
    for m in range(2):
        norm = _norm(degs_ref[m, 1, :])
        term = (agg_ref[m] * norm[:N, None]) * beta_ref[m]
        acc = term if acc is None else acc + term
    out_ref[...] = acc


def kernel(x, edge_index_0, edge_index_1, W1, b1, W2):
    edges = jnp.concatenate(
        [edge_index_0[0], edge_index_0[1], edge_index_1[0], edge_index_1[1]])
    ar = jnp.arange(PADE, dtype=jnp.int32)
    pad_src = ar % N
    pad_dst = TRASH + (ar % 128)
    src_agg = jnp.concatenate(
        [edge_index_0[0], pad_src, edge_index_1[0] + N, pad_src + N])
    dst_agg = jnp.concatenate(
        [edge_index_0[1], pad_dst, edge_index_1[1], pad_dst])
    degs = _deg_kernel(edges).reshape(2, 2, NPAD)
    msg = pl.pallas_call(
        _msg_body,
        out_shape=jax.ShapeDtypeStruct((2, N, D), jnp.float32),
    )(x, degs)
    agg = _agg_kernel(src_agg, dst_agg, msg.reshape(2 * N, D)).reshape(2, N, D)
    beta = pl.pallas_call(
        _stats_body,
        out_shape=jax.ShapeDtypeStruct((2,), jnp.float32),
        out_specs=pl.BlockSpec(memory_space=pltpu.SMEM),
    )(agg, degs, W1, b1, W2.T)
    out = pl.pallas_call(
        _comb_body,
        in_specs=[
            pl.BlockSpec(memory_space=pltpu.VMEM),
            pl.BlockSpec(memory_space=pltpu.VMEM),
            pl.BlockSpec(memory_space=pltpu.SMEM),
        ],
        out_shape=jax.ShapeDtypeStruct((N, D), jnp.float32),
    )(agg, degs, beta)
    return out


# speedup vs baseline: 2.6943x; 1.0070x over previous
import functools

import jax
import jax.numpy as jnp
from jax import lax
from jax.experimental import pallas as pl
from jax.experimental.pallas import tpu as pltpu
from jax.experimental.pallas import tpu_sc as plsc

N = 10000
D = 128
E = 320000
NPAD = 10240
HR = NPAD // 128
NS = 16
EPT = E // NS
CH = 112
CPT = 180
SPT = CPT * CH
PADE = NS * SPT - E
TRASH = N

_mesh = plsc.VectorSubcoreMesh(core_axis_name="c", subcore_axis_name="s")
_sc_params = pltpu.CompilerParams(needs_layout_passes=False)


@functools.partial(
    pl.kernel,
    out_type=jax.ShapeDtypeStruct((2 * 2 * HR, 128), jnp.float32),
    mesh=_mesh,
    scratch_types=[
        pltpu.VMEM((EPT,), jnp.int32),
        pltpu.VMEM((EPT,), jnp.int32),
        pltpu.VMEM((2 * HR, 128), jnp.float32),
        pltpu.VMEM((HR,), jnp.int32),
        pltpu.VMEM((HR,), jnp.int32),
        pltpu.VMEM_SHARED((2 * HR, 128), jnp.float32),
    ],
    compiler_params=_sc_params,
)
def _deg_kernel(edges_hbm, out_hbm, sidx, didx, hist, rs, rd, shist):
    c = lax.axis_index("c")
    s = lax.axis_index("s")
    zero16 = jnp.zeros((16,), jnp.float32)

    def zbody(i, _):
        for k in range(8):
            hist[i, pl.ds(k * 16, 16)] = zero16
        return 0
    lax.fori_loop(0, 2 * HR, zbody, 0)

    iota = lax.iota(jnp.int32, 16)
    for j in range(HR // 16):
        rs[pl.ds(j * 16, 16)] = iota + (j * 16)
        rd[pl.ds(j * 16, 16)] = iota + (HR + j * 16)

    @pl.when(s == 0)
    def _():
        pltpu.sync_copy(hist, shist)

    sbase = (2 * c) * E + s * EPT
    dbase = (2 * c + 1) * E + s * EPT
    pltpu.sync_copy(edges_hbm.at[pl.ds(sbase, EPT)], sidx)
    pltpu.sync_copy(edges_hbm.at[pl.ds(dbase, EPT)], didx)

    ones = jnp.ones((16,), jnp.float32)

    def count(idx_ref, row_off):
        def body(i, _):
            for k in range(4):
                v = idx_ref[pl.ds(i * 64 + k * 16, 16)]
                r = lax.shift_right_logical(v, 7) + row_off
                col = jnp.bitwise_and(v, 127)
                plsc.addupdate_scatter(hist, [r, col], ones)
            return 0
        lax.fori_loop(0, EPT // 64, body, 0)

    count(sidx, 0)
    count(didx, HR)
    plsc.subcore_barrier()
    pltpu.sync_copy(hist.at[pl.ds(0, HR)], shist.at[rs], add=True)
    pltpu.sync_copy(hist.at[pl.ds(HR, HR)], shist.at[rd], add=True)
    plsc.subcore_barrier()

    @pl.when(s < 10)
    def _():
        pltpu.sync_copy(shist.at[pl.ds(s * 16, 16)],
                        out_hbm.at[pl.ds(c * (2 * HR) + s * 16, 16)])


@functools.partial(
    pl.kernel,
    out_type=jax.ShapeDtypeStruct((2 * N, 128), jnp.float32),
    mesh=_mesh,
    scratch_types=[
        [pltpu.VMEM((CH,), jnp.int32)] * 6,
        [pltpu.VMEM((CH,), jnp.int32)] * 6,
        [pltpu.VMEM((CH, 128), jnp.float32)] * 3,
        pltpu.VMEM((16, 128), jnp.float32),
        pltpu.VMEM_SHARED((N + 32, 128), jnp.float32),
        [pltpu.SemaphoreType.DMA] * 6,
        [pltpu.SemaphoreType.DMA] * 3,
    ],
    compiler_params=_sc_params,
)
def _agg_kernel(src_hbm, dst_hbm, msg_hbm, out_hbm,
                sidx, didx, rows, zbuf, agg, sem_i, sem_g):
    c = lax.axis_index("c")
    s = lax.axis_index("s")

    ebase = (c * NS + s) * SPT

    def start_idx(k, p):
        off = ebase + k * CH
        pltpu.async_copy(src_hbm.at[pl.ds(off, CH)], sidx[p], sem_i[p])
        pltpu.async_copy(dst_hbm.at[pl.ds(off, CH)], didx[p], sem_i[p])

    def wait_idx(p):
        pltpu.make_async_copy(src_hbm.at[pl.ds(0, CH)], sidx[p], sem_i[p]).wait()
        pltpu.make_async_copy(dst_hbm.at[pl.ds(0, CH)], didx[p], sem_i[p]).wait()

    def sg(p, b):
        pltpu.async_copy(msg_hbm.at[sidx[p]], rows[b], sem_g[b])

    def wg(b):
        pltpu.make_async_copy(msg_hbm.at[sidx[0]], rows[b], sem_g[b]).wait()

    def ssync(p, b):
        pltpu.sync_copy(rows[b], agg.at[didx[p]], add=True)

    for p in range(6):
        start_idx(p, p)

    zero16 = jnp.zeros((16,), jnp.float32)
    for r in range(16):
        for k in range(8):
            zbuf[r, pl.ds(k * 16, 16)] = zero16

    rpt = 624
    zb_base = s * rpt

    def zb(i, _):
        pltpu.sync_copy(zbuf, agg.at[pl.ds(zb_base + i * 16, 16)])
        return 0
    lax.fori_loop(0, rpt // 16, zb, 0)

    @pl.when(s == 0)
    def _():
        pltpu.sync_copy(zbuf, agg.at[pl.ds(NS * rpt, 16)])
    plsc.subcore_barrier()

    wait_idx(0)
    sg(0, 0)
    wait_idx(1)
    sg(1, 1)

    NJ = CPT // 6
    def pipe(j, _):
        k = 6 * j
        for m in range(6):
            pg = (m + 2) % 6
            if m < 4:
                wait_idx(pg)
                sg(pg, (m + 2) % 3)
            else:
                @pl.when(j < NJ - 1)
                def _(pg=pg, m=m):
                    wait_idx(pg)
                    sg(pg, (m + 2) % 3)
            wg(m % 3)
            ssync(m, m % 3)

            @pl.when(j < NJ - 1)
            def _(m=m):
                start_idx(k + m + 6, m)
        return 0
    lax.fori_loop(0, NJ, pipe, 0)

    plsc.subcore_barrier()
    pltpu.sync_copy(agg.at[pl.ds(s * rpt, rpt)],
                    out_hbm.at[pl.ds(c * N + s * rpt, rpt)])

    @pl.when(s == 0)
    def _():
        pltpu.sync_copy(agg.at[pl.ds(NS * rpt, 16)],
                        out_hbm.at[pl.ds(c * N + NS * rpt, 16)])


def _norm(deg):
    return jnp.where(deg > 0, lax.rsqrt(jnp.maximum(deg, 1e-12)), 0.0)


def _msg_body(x_ref, degs_ref, out_ref):
    x = x_ref[...]
    for m in range(2):
        norm = _norm(degs_ref[m, 0, :])
        out_ref[m] = x * norm[:N, None]


def _stats_body(agg_ref, degs_ref, w1_ref, b1_ref, w2t_ref, beta_ref):
    acc = []
    for m in range(2):
        norm = _norm(degs_ref[m, 1, :])
        h = agg_ref[m] * norm[:N, None]
        t = jnp.tanh(
            jnp.dot(h, w1_ref[...], preferred_element_type=jnp.float32)
            + b1_ref[...][None, :])
        acc.append(jnp.sum(t * w2t_ref[...]) / N)
    w0, w1 = acc
    mx = jnp.maximum(w0, w1)
    e0 = jnp.exp(w0 - mx)
    e1 = jnp.exp(w1 - mx)
    beta_ref[0] = e0 / (e0 + e1)
    beta_ref[1] = e1 / (e0 + e1)


def _comb_body(agg_ref, degs_ref, beta_ref, out_ref):
    acc = None
    for m in range(2):
        norm = _norm(degs_ref[m, 1, :])
        term = (agg_ref[m] * norm[:N, None]) * beta_ref[m]
        acc = term if acc is None else acc + term
    out_ref[...] = acc


def kernel(x, edge_index_0, edge_index_1, W1, b1, W2):
    edges = jnp.concatenate(
        [edge_index_0[0], edge_index_0[1], edge_index_1[0], edge_index_1[1]])
    ar = jnp.arange(PADE, dtype=jnp.int32)
    pad_src = ar % N
    pad_dst = TRASH + (ar % 128)
    src_agg = jnp.concatenate(
        [edge_index_0[0], pad_src, edge_index_1[0] + N, pad_src + N])
    dst_agg = jnp.concatenate(
        [edge_index_0[1], pad_dst, edge_index_1[1], pad_dst])
    degs = _deg_kernel(edges).reshape(2, 2, NPAD)
    msg = pl.pallas_call(
        _msg_body,
        out_shape=jax.ShapeDtypeStruct((2, N, D), jnp.float32),
    )(x, degs)
    agg = _agg_kernel(src_agg, dst_agg, msg.reshape(2 * N, D)).reshape(2, N, D)
    beta = pl.pallas_call(
        _stats_body,
        out_shape=jax.ShapeDtypeStruct((2,), jnp.float32),
        out_specs=pl.BlockSpec(memory_space=pltpu.SMEM),
    )(agg, degs, W1, b1, W2.T)
    out = pl.pallas_call(
        _comb_body,
        in_specs=[
            pl.BlockSpec(memory_space=pltpu.VMEM),
            pl.BlockSpec(memory_space=pltpu.VMEM),
            pl.BlockSpec(memory_space=pltpu.SMEM),
        ],
        out_shape=jax.ShapeDtypeStruct((N, D), jnp.float32),
    )(agg, degs, beta)
    return out
